# baseline (speedup 1.0000x reference)
import functools

import numpy as np
import jax
import jax.numpy as jnp
from jax import lax
from jax.experimental import pallas as pl
from jax.experimental.pallas import tpu as pltpu
from jax.experimental.pallas import tpu_sc as plsc

B = 4096
F = 26
V = 1000
D = 16
C = F * (F - 1) // 2
RED = C // 2
H1, H2 = 1024, 512

_IU, _JU = np.triu_indices(F, k=1)

NC, NS = 2, 16
NW = NC * NS
CP = 328
RL = CP // 8
BB = 128
NBT = B // BB
IDXROW = 128
CG = 8
ROWS = CG * IDXROW
NCH_AB = NBT * RL
ITER_AB = NCH_AB // NW
NCH_L = B * F // ROWS
ITER_L = -(-NCH_L // NW)

def _sc_gather_body(ffm_hbm, lin_hbm, idxa_hbm, idxb_hbm, idxl_hbm,
               ema_hbm, emb_hbm, linv_hbm,
               idxa_v, idxb_v, idxl_v, rowsa_v, rowsb_v, rowsl_v,
               sema, semb, seml):
    wid = lax.axis_index("s") * NC + lax.axis_index("c")

    def chunk(k, carry):
        m = k * NW + wid
        pltpu.sync_copy(idxa_hbm.at[m], idxa_v)
        pltpu.sync_copy(idxb_hbm.at[m], idxb_v)
        cps = []
        for g in range(CG):
            cps.append(pltpu.async_copy(
                ffm_hbm.at[idxa_v.at[g]],
                rowsa_v.at[pl.ds(g * IDXROW, IDXROW)], sema))
            cps.append(pltpu.async_copy(
                ffm_hbm.at[idxb_v.at[g]],
                rowsb_v.at[pl.ds(g * IDXROW, IDXROW)], semb))
        for cp in cps:
            cp.wait()
        pltpu.sync_copy(rowsa_v, ema_hbm.at[pl.ds(m * ROWS, ROWS)])
        pltpu.sync_copy(rowsb_v, emb_hbm.at[pl.ds(m * ROWS, ROWS)])
        return carry

    lax.fori_loop(0, ITER_AB, chunk, 0)

    def lchunk(k, carry):
        m = k * NW + wid

        @pl.when(m < NCH_L)
        def _():
            pltpu.sync_copy(idxl_hbm.at[m], idxl_v)
            cps = [pltpu.async_copy(lin_hbm.at[idxl_v.at[g]],
                                    rowsl_v.at[pl.ds(g * IDXROW, IDXROW)], seml)
                   for g in range(CG)]
            for cp in cps:
                cp.wait()
            pltpu.sync_copy(rowsl_v, linv_hbm.at[pl.ds(m * ROWS, ROWS)])

        return carry

    lax.fori_loop(0, ITER_L, lchunk, 0)


@functools.lru_cache(maxsize=1)
def _sc_gather():
    mesh = plsc.VectorSubcoreMesh(core_axis_name="c", subcore_axis_name="s",
                                  num_cores=NC, num_subcores=NS)
    return pl.kernel(
        _sc_gather_body,
        out_type=(jax.ShapeDtypeStruct((B * CP, D), jnp.float32),
                  jax.ShapeDtypeStruct((B * CP, D), jnp.float32),
                  jax.ShapeDtypeStruct((B * F, D), jnp.float32)),
        mesh=mesh,
        compiler_params=pltpu.CompilerParams(use_tc_tiling_on_sc=False),
        scratch_types=[
            pltpu.VMEM((CG, IDXROW), jnp.int32),
            pltpu.VMEM((CG, IDXROW), jnp.int32),
            pltpu.VMEM((CG, IDXROW), jnp.int32),
            pltpu.VMEM((ROWS, D), jnp.float32),
            pltpu.VMEM((ROWS, D), jnp.float32),
            pltpu.VMEM((ROWS, D), jnp.float32),
            pltpu.SemaphoreType.DMA,
            pltpu.SemaphoreType.DMA,
            pltpu.SemaphoreType.DMA,
        ],
    )


SPF = RL * IDXROW

_GBIG_NP = np.zeros((SPF, CP), dtype=np.float32)
_GBIG_NP[np.arange(SPF), np.arange(SPF) // D] = 1.0
_GT_NP = np.ascontiguousarray(_GBIG_NP.T)
_GT_NP[C:] = 0.0


def _tc_body(ema_ref, emb_ref, linv_ref, cw_ref, cb_ref, g_ref, gt_ref,
             ew1_ref, eb1_ref, ew2_ref, eb2_ref,
             w1_ref, b1_ref, w2_ref, b2_ref, w3_ref, c0_ref, out_ref,
             em_scr):
    f32 = jnp.float32
    for r in range(RL):
        em_scr[:, pl.ds(r * IDXROW, IDXROW)] = (
            ema_ref[pl.ds(r * BB, BB), :] * emb_ref[pl.ds(r * BB, BB), :])
    em = em_scr[...]
    emw = (em * cw_ref[...]).astype(jnp.bfloat16)
    dcomp = jnp.dot(emw, g_ref[...],
                    preferred_element_type=f32) + cb_ref[...]
    t = jnp.maximum(jnp.dot(dcomp, ew1_ref[...],
                            preferred_element_type=f32) + eb1_ref[...], 0.0)
    s = jnp.maximum(jnp.dot(t, ew2_ref[...],
                            preferred_element_type=f32) + eb2_ref[...], 0.0)
    sexp = jnp.dot(s.astype(jnp.bfloat16), gt_ref[...],
                   preferred_element_type=f32)
    aem = (em * sexp).astype(jnp.bfloat16)
    h = jnp.maximum(jnp.dot(aem, w1_ref[...],
                            preferred_element_type=f32) + b1_ref[...], 0.0)
    h = jnp.maximum(jnp.dot(h.astype(jnp.bfloat16), w2_ref[...],
                            preferred_element_type=f32) + b2_ref[...], 0.0)
    ylin = jnp.sum(linv_ref[...], axis=1, keepdims=True)
    y = jnp.sum(h * w3_ref[...], axis=1, keepdims=True) + c0_ref[...] + ylin
    out_ref[...] = 1.0 / (1.0 + jnp.exp(-y))


def _tc_specs():
    zero = lambda i: (0, 0)
    row = lambda i: (i, 0)
    in_specs = [
        pl.BlockSpec((RL * BB, IDXROW), row),
        pl.BlockSpec((RL * BB, IDXROW), row),
        pl.BlockSpec((BB, F * D), row),
        pl.BlockSpec((1, SPF), zero),
        pl.BlockSpec((1, CP), zero),
        pl.BlockSpec((SPF, CP), zero),
        pl.BlockSpec((CP, SPF), zero),
        pl.BlockSpec((CP, RED), zero),
        pl.BlockSpec((1, RED), zero),
        pl.BlockSpec((RED, CP), zero),
        pl.BlockSpec((1, CP), zero),
        pl.BlockSpec((SPF, H1), zero),
        pl.BlockSpec((1, H1), zero),
        pl.BlockSpec((H1, H2), zero),
        pl.BlockSpec((1, H2), zero),
        pl.BlockSpec((1, H2), zero),
        pl.BlockSpec((1, 1), zero),
    ]
    out_spec = pl.BlockSpec((BB, 1), row)
    return in_specs, out_spec


def _mk_idx_consts():
    iu_p = np.concatenate([_IU, np.zeros(CP - C, dtype=np.int64)])
    ju_p = np.concatenate([_JU, np.zeros(CP - C, dtype=np.int64)])
    i_ = np.arange(NBT)[:, None, None, None]
    r_ = np.arange(RL)[None, :, None, None]
    bl = np.arange(BB)[None, None, :, None]
    c8 = np.arange(8)[None, None, None, :]
    b = (i_ * BB + bl) + 0 * (r_ + c8)
    c = (r_ * 8 + c8) + 0 * (i_ + bl)
    shp = (NCH_AB, CG, IDXROW)
    ga = np.broadcast_to(b * F + iu_p[c], (NBT, RL, BB, 8)).reshape(shp)
    ca = np.broadcast_to(iu_p[c] * (V * F) + ju_p[c], (NBT, RL, BB, 8)).reshape(shp)
    gb = np.broadcast_to(b * F + ju_p[c], (NBT, RL, BB, 8)).reshape(shp)
    cb = np.broadcast_to(ju_p[c] * (V * F) + iu_p[c], (NBT, RL, BB, 8)).reshape(shp)
    return (ga.astype(np.int32), ca.astype(np.int32),
            gb.astype(np.int32), cb.astype(np.int32))


_GA_NP, _CA_NP, _GB_NP, _CB_NP = _mk_idx_consts()


def kernel(x, lin_tables, ffm_tables, compose_w, compose_b,
           exc_w1, exc_b1, exc_w2, exc_b2,
           mlp_w1, mlp_b1, mlp_w2, mlp_b2, mlp_w3, mlp_b3, b_global):
    iu = jnp.asarray(_IU, dtype=jnp.int32)
    ju = jnp.asarray(_JU, dtype=jnp.int32)
    ffm_g = jax.lax.optimization_barrier(
        ffm_tables.reshape(F, V * F // 8, 8, D).reshape(F * V * F // 8, 8 * D))
    ffm_flat = ffm_g.reshape(F * V * F, D)
    lin_pad = jnp.pad(lin_tables.reshape(F * V, 1), ((0, 0), (0, D - 1)))
    xflat = x.reshape(B * F)
    idxa2 = jnp.asarray(_CA_NP) + jnp.take(xflat, jnp.asarray(_GA_NP)) * F
    idxb2 = jnp.asarray(_CB_NP) + jnp.take(xflat, jnp.asarray(_GB_NP)) * F
    idxl = (jnp.arange(F, dtype=jnp.int32) * V)[None, :] + x
    idxl2 = idxl.reshape(NCH_L, CG, IDXROW)

    ema, emb, linv = _sc_gather()(ffm_flat, lin_pad, idxa2, idxb2, idxl2)

    ema2 = ema.reshape(B * CP * D // IDXROW, IDXROW)
    emb2 = emb.reshape(B * CP * D // IDXROW, IDXROW)
    linv2 = linv.reshape(B, F * D)
    c0 = (mlp_b3[0] + b_global[0]).reshape(1, 1)

    cwp = jnp.pad(compose_w.reshape(C * D), (0, SPF - C * D)).reshape(1, SPF)
    cbp = jnp.pad(compose_b, (0, CP - C)).reshape(1, CP)
    ew1p = jnp.pad(exc_w1, ((0, CP - C), (0, 0)))
    ew2p = jnp.pad(exc_w2, ((0, 0), (0, CP - C)))
    eb2p = jnp.pad(exc_b2, (0, CP - C)).reshape(1, CP)
    w1p = jnp.pad(mlp_w1, ((0, SPF - C * D), (0, 0))).astype(jnp.bfloat16)

    in_specs, out_spec = _tc_specs()
    out = pl.pallas_call(
        _tc_body,
        grid=(NBT,),
        in_specs=in_specs,
        out_specs=out_spec,
        out_shape=jax.ShapeDtypeStruct((B, 1), jnp.float32),
        scratch_shapes=[pltpu.VMEM((BB, SPF), jnp.float32)],
    )(ema2, emb2, linv2,
      cwp, cbp,
      jnp.asarray(_GBIG_NP, dtype=jnp.bfloat16),
      jnp.asarray(_GT_NP, dtype=jnp.bfloat16),
      ew1p, exc_b1.reshape(1, RED), ew2p, eb2p,
      w1p, mlp_b1.reshape(1, H1),
      mlp_w2.astype(jnp.bfloat16), mlp_b2.reshape(1, H2),
      mlp_w3.reshape(1, H2), c0)
    return out.reshape(B)


# speedup vs baseline: 35.3759x; 35.3759x over previous
import functools

import numpy as np
import jax
import jax.numpy as jnp
from jax import lax
from jax.experimental import pallas as pl
from jax.experimental.pallas import tpu as pltpu
from jax.experimental.pallas import tpu_sc as plsc

B = 4096
F = 26
V = 1000
D = 16
C = F * (F - 1) // 2
RED = C // 2
H1, H2 = 1024, 512

_IU, _JU = np.triu_indices(F, k=1)

NC, NS = 2, 16
NW = NC * NS
CP = 328
RL = CP // 8
BB = 128
NBT = B // BB
IDXROW = 128
CG = 8
ROWS = CG * IDXROW
NCH_AB = NBT * RL
ITER_AB = NCH_AB // NW
NCH_L = B * F // ROWS
ITER_L = -(-NCH_L // NW)

def _sc_gather_body(ffm_hbm, lin_hbm, idxa_hbm, idxb_hbm, idxl_hbm,
               ema_hbm, emb_hbm, linv_hbm,
               idxa_v, idxb_v, idxl_v, rowsa_v, rowsb_v, rowsl_v,
               sema, semb, seml):
    wid = lax.axis_index("s") * NC + lax.axis_index("c")

    def chunk(k, carry):
        m = k * NW + wid
        pltpu.sync_copy(idxa_hbm.at[m], idxa_v)
        pltpu.sync_copy(idxb_hbm.at[m], idxb_v)
        cps = []
        for g in range(CG):
            cps.append(pltpu.async_copy(
                ffm_hbm.at[idxa_v.at[g]],
                rowsa_v.at[pl.ds(g * IDXROW, IDXROW)], sema))
            cps.append(pltpu.async_copy(
                ffm_hbm.at[idxb_v.at[g]],
                rowsb_v.at[pl.ds(g * IDXROW, IDXROW)], semb))
        for cp in cps:
            cp.wait()
        pltpu.sync_copy(rowsa_v, ema_hbm.at[pl.ds(m * ROWS, ROWS)])
        pltpu.sync_copy(rowsb_v, emb_hbm.at[pl.ds(m * ROWS, ROWS)])
        return carry

    lax.fori_loop(0, ITER_AB, chunk, 0)

    def lchunk(k, carry):
        m = k * NW + wid

        @pl.when(m < NCH_L)
        def _():
            pltpu.sync_copy(idxl_hbm.at[m], idxl_v)
            cps = [pltpu.async_copy(lin_hbm.at[idxl_v.at[g]],
                                    rowsl_v.at[pl.ds(g * IDXROW, IDXROW)], seml)
                   for g in range(CG)]
            for cp in cps:
                cp.wait()
            pltpu.sync_copy(rowsl_v, linv_hbm.at[pl.ds(m * ROWS, ROWS)])

        return carry

    lax.fori_loop(0, ITER_L, lchunk, 0)


@functools.lru_cache(maxsize=1)
def _sc_gather():
    mesh = plsc.VectorSubcoreMesh(core_axis_name="c", subcore_axis_name="s",
                                  num_cores=NC, num_subcores=NS)
    return pl.kernel(
        _sc_gather_body,
        out_type=(jax.ShapeDtypeStruct((B * CP, D), jnp.float32),
                  jax.ShapeDtypeStruct((B * CP, D), jnp.float32),
                  jax.ShapeDtypeStruct((B * F, D), jnp.float32)),
        mesh=mesh,
        compiler_params=pltpu.CompilerParams(use_tc_tiling_on_sc=False),
        scratch_types=[
            pltpu.VMEM((CG, IDXROW), jnp.int32),
            pltpu.VMEM((CG, IDXROW), jnp.int32),
            pltpu.VMEM((CG, IDXROW), jnp.int32),
            pltpu.VMEM((ROWS, D), jnp.float32),
            pltpu.VMEM((ROWS, D), jnp.float32),
            pltpu.VMEM((ROWS, D), jnp.float32),
            pltpu.SemaphoreType.DMA,
            pltpu.SemaphoreType.DMA,
            pltpu.SemaphoreType.DMA,
        ],
    )


SPF = RL * IDXROW

_GBIG_NP = np.zeros((SPF, CP), dtype=np.float32)
_GBIG_NP[np.arange(SPF), np.arange(SPF) // D] = 1.0
_GT_NP = np.ascontiguousarray(_GBIG_NP.T)
_GT_NP[C:] = 0.0


def _tc_body(ema_ref, emb_ref, linv_ref, cw_ref, cb_ref, g_ref, gt_ref,
             ew1_ref, eb1_ref, ew2_ref, eb2_ref,
             w1_ref, b1_ref, w2_ref, b2_ref, w3_ref, c0_ref, out_ref,
             em_scr):
    f32 = jnp.float32
    for r in range(RL):
        em_scr[:, pl.ds(r * IDXROW, IDXROW)] = (
            ema_ref[pl.ds(r * BB, BB), :] * emb_ref[pl.ds(r * BB, BB), :])
    em = em_scr[...]
    emw = (em * cw_ref[...]).astype(jnp.bfloat16)
    dcomp = jnp.dot(emw, g_ref[...],
                    preferred_element_type=f32) + cb_ref[...]
    t = jnp.maximum(jnp.dot(dcomp, ew1_ref[...],
                            preferred_element_type=f32) + eb1_ref[...], 0.0)
    s = jnp.maximum(jnp.dot(t, ew2_ref[...],
                            preferred_element_type=f32) + eb2_ref[...], 0.0)
    sexp = jnp.dot(s.astype(jnp.bfloat16), gt_ref[...],
                   preferred_element_type=f32)
    aem = (em * sexp).astype(jnp.bfloat16)
    h = jnp.maximum(jnp.dot(aem, w1_ref[...],
                            preferred_element_type=f32) + b1_ref[...], 0.0)
    h = jnp.maximum(jnp.dot(h.astype(jnp.bfloat16), w2_ref[...],
                            preferred_element_type=f32) + b2_ref[...], 0.0)
    ylin = jnp.sum(linv_ref[...], axis=1, keepdims=True)
    y = jnp.sum(h * w3_ref[...], axis=1, keepdims=True) + c0_ref[...] + ylin
    out_ref[...] = 1.0 / (1.0 + jnp.exp(-y))


def _tc_specs():
    zero = lambda i: (0, 0)
    row = lambda i: (i, 0)
    in_specs = [
        pl.BlockSpec((RL * BB, IDXROW), row),
        pl.BlockSpec((RL * BB, IDXROW), row),
        pl.BlockSpec((BB, F * D), row),
        pl.BlockSpec((1, SPF), zero),
        pl.BlockSpec((1, CP), zero),
        pl.BlockSpec((SPF, CP), zero),
        pl.BlockSpec((CP, SPF), zero),
        pl.BlockSpec((CP, RED), zero),
        pl.BlockSpec((1, RED), zero),
        pl.BlockSpec((RED, CP), zero),
        pl.BlockSpec((1, CP), zero),
        pl.BlockSpec((SPF, H1), zero),
        pl.BlockSpec((1, H1), zero),
        pl.BlockSpec((H1, H2), zero),
        pl.BlockSpec((1, H2), zero),
        pl.BlockSpec((1, H2), zero),
        pl.BlockSpec((1, 1), zero),
    ]
    out_spec = pl.BlockSpec((BB, 1), row)
    return in_specs, out_spec


_IU_PAD = np.concatenate([_IU, np.zeros(CP - C, dtype=_IU.dtype)])
_JU_PAD = np.concatenate([_JU, np.zeros(CP - C, dtype=_JU.dtype)])


def _arrange_idx(idx_pad):
    a = idx_pad.reshape(NBT, BB, RL, 8)
    a = jnp.swapaxes(a, 1, 2)
    return a.reshape(NCH_AB, CG, IDXROW)


def kernel(x, lin_tables, ffm_tables, compose_w, compose_b,
           exc_w1, exc_b1, exc_w2, exc_b2,
           mlp_w1, mlp_b1, mlp_w2, mlp_b2, mlp_w3, mlp_b3, b_global):
    iu = jnp.asarray(_IU, dtype=jnp.int32)
    ju = jnp.asarray(_JU, dtype=jnp.int32)
    ffm_g = jax.lax.optimization_barrier(
        ffm_tables.reshape(F, V * F // 8, 8, D).reshape(F * V * F // 8, 8 * D))
    ffm_flat = ffm_g.reshape(F * V * F, D)
    lin_pad = jnp.pad(lin_tables.reshape(F * V, 1), ((0, 0), (0, D - 1)))
    iup = jnp.asarray(_IU_PAD, dtype=jnp.int32)
    jup = jnp.asarray(_JU_PAD, dtype=jnp.int32)
    idxa = (iup * (V * F) + jup)[None, :] + jnp.take(x, iup, axis=1) * F
    idxb = (jup * (V * F) + iup)[None, :] + jnp.take(x, jup, axis=1) * F
    idxa2 = _arrange_idx(idxa)
    idxb2 = _arrange_idx(idxb)
    idxl = (jnp.arange(F, dtype=jnp.int32) * V)[None, :] + x
    idxl2 = idxl.reshape(NCH_L, CG, IDXROW)

    ema, emb, linv = _sc_gather()(ffm_flat, lin_pad, idxa2, idxb2, idxl2)

    ema2 = ema.reshape(B * CP * D // IDXROW, IDXROW)
    emb2 = emb.reshape(B * CP * D // IDXROW, IDXROW)
    linv2 = linv.reshape(B, F * D)
    c0 = (mlp_b3[0] + b_global[0]).reshape(1, 1)

    cwp = jnp.pad(compose_w.reshape(C * D), (0, SPF - C * D)).reshape(1, SPF)
    cbp = jnp.pad(compose_b, (0, CP - C)).reshape(1, CP)
    ew1p = jnp.pad(exc_w1, ((0, CP - C), (0, 0)))
    ew2p = jnp.pad(exc_w2, ((0, 0), (0, CP - C)))
    eb2p = jnp.pad(exc_b2, (0, CP - C)).reshape(1, CP)
    w1p = jnp.pad(mlp_w1, ((0, SPF - C * D), (0, 0))).astype(jnp.bfloat16)

    in_specs, out_spec = _tc_specs()
    out = pl.pallas_call(
        _tc_body,
        grid=(NBT,),
        in_specs=in_specs,
        out_specs=out_spec,
        out_shape=jax.ShapeDtypeStruct((B, 1), jnp.float32),
        scratch_shapes=[pltpu.VMEM((BB, SPF), jnp.float32)],
    )(ema2, emb2, linv2,
      cwp, cbp,
      jnp.asarray(_GBIG_NP, dtype=jnp.bfloat16),
      jnp.asarray(_GT_NP, dtype=jnp.bfloat16),
      ew1p, exc_b1.reshape(1, RED), ew2p, eb2p,
      w1p, mlp_b1.reshape(1, H1),
      mlp_w2.astype(jnp.bfloat16), mlp_b2.reshape(1, H2),
      mlp_w3.reshape(1, H2), c0)
    return out.reshape(B)
